# baseline (speedup 1.0000x reference)
import functools

import jax
import jax.numpy as jnp
from jax import lax
from jax.experimental import pallas as pl
from jax.experimental.pallas import tpu as pltpu
from jax.experimental.pallas import tpu_sc as plsc

_N = 1_000_000
_C = 100
_B = 2048
_GRID = -(-_N // _B)

_NW = 32
_CHUNK = 31248
_TAIL = _N - (_NW - 1) * _CHUNK
_STEPS = _CHUNK // 16
_TSTEPS = _TAIL // 16
_HB = 128
_HSIZE = 16 * _HB


def _amax_body(yp_ref, yt_ref, out_ref):
    x = yp_ref[...]
    xt = jnp.swapaxes(x, 0, 1)
    ytv = yt_ref[...][None, :]
    idxs = lax.broadcasted_iota(jnp.int32, (_C, _B), 0)
    neg = jnp.float32(-jnp.inf)
    xv = jnp.max(jnp.where(idxs == ytv, xt, neg), axis=0, keepdims=True)
    m = jnp.max(xt, axis=0, keepdims=True)
    correct = xv >= m
    out_ref[...] = jnp.where(correct, ytv, _C).astype(jnp.int32)[0]


def _sc_hist_body(src_hbm, out_hbm, src_v, hist_v):
    wid = lax.axis_index("s") * 2 + lax.axis_index("c")
    base = pl.multiple_of(wid * _CHUNK, 16)

    def _zero(j, _):
        hist_v[pl.ds(j * 16, 16)] = jnp.zeros((16,), jnp.int32)
        return 0

    lax.fori_loop(0, _HSIZE // 16, _zero, 0)

    pltpu.sync_copy(src_hbm.at[pl.ds(base, _CHUNK)], src_v.at[pl.ds(0, _CHUNK)])

    @pl.when(wid == _NW - 1)
    def _tail_copy():
        off = _N - (_TAIL - _CHUNK)
        pltpu.sync_copy(
            src_hbm.at[pl.ds(off, _TAIL - _CHUNK)],
            src_v.at[pl.ds(_CHUNK, _TAIL - _CHUNK)],
        )

    lanes = lax.iota(jnp.int32, 16) * _HB
    ones = jnp.ones((16,), jnp.int32)

    def _step(i, _):
        b = src_v[pl.ds(i * 16, 16)]
        plsc.addupdate_scatter(hist_v, [lanes + b], ones)
        return 0

    lax.fori_loop(0, _STEPS, _step, 0)

    @pl.when(wid == _NW - 1)
    def _tail_steps():
        lax.fori_loop(_STEPS, _TSTEPS, _step, 0)

    pltpu.sync_copy(hist_v, out_hbm.at[wid])


_sc_hist = functools.partial(
    pl.kernel,
    mesh=plsc.VectorSubcoreMesh(core_axis_name="c", subcore_axis_name="s"),
    out_type=jax.ShapeDtypeStruct((_NW, _HSIZE), jnp.int32),
    scratch_types=[
        pltpu.VMEM((_TAIL,), jnp.int32),
        pltpu.VMEM((_HSIZE,), jnp.int32),
    ],
    compiler_params=pltpu.CompilerParams(needs_layout_passes=False),
)


def _fin_body(hp_ref, ht_ref, w_ref, out_ref):
    sp = jnp.sum(hp_ref[...], axis=0, keepdims=True)
    st = jnp.sum(ht_ref[...], axis=0, keepdims=True)
    cp = jnp.zeros((1, _HB), jnp.int32)
    ct = jnp.zeros((1, _HB), jnp.int32)
    for l in range(16):
        cp = cp + sp[0:1, l * _HB : (l + 1) * _HB]
        ct = ct + st[0:1, l * _HB : (l + 1) * _HB]
    ctc = ct[0:1, :_C]
    acc = jnp.where(
        ctc > 0,
        cp[0:1, :_C].astype(jnp.float32) / jnp.maximum(ctc, 1).astype(jnp.float32),
        0.0,
    )
    w = w_ref[...]
    val = jnp.sum(acc * w) / jnp.sum(w)
    out_ref[...] = jnp.broadcast_to(val, (1, 1))


def kernel(y_pred, y_true, weights):
    yt32 = y_true.astype(jnp.int32)

    hist_t = _sc_hist(_sc_hist_body)(yt32)

    masked_bin = pl.pallas_call(
        _amax_body,
        grid=(_GRID,),
        in_specs=[
            pl.BlockSpec((_B, _C), lambda i: (i, 0)),
            pl.BlockSpec((_B,), lambda i: (i,)),
        ],
        out_specs=pl.BlockSpec((_B,), lambda i: (i,)),
        out_shape=jax.ShapeDtypeStruct((_N,), jnp.int32),
    )(y_pred, yt32)

    hist_p = _sc_hist(_sc_hist_body)(masked_bin)

    out = pl.pallas_call(
        _fin_body,
        in_specs=[
            pl.BlockSpec((_NW, _HSIZE), lambda: (0, 0)),
            pl.BlockSpec((_NW, _HSIZE), lambda: (0, 0)),
            pl.BlockSpec((1, _C), lambda: (0, 0)),
        ],
        out_specs=pl.BlockSpec((1, 1), lambda: (0, 0)),
        out_shape=jax.ShapeDtypeStruct((1, 1), jnp.float32),
    )(hist_p, hist_t, weights.reshape(1, _C))
    return out.reshape(())
